# Initial kernel scaffold; baseline (speedup 1.0000x reference)
import jax
import jax.numpy as jnp
from jax.experimental import pallas as pl


def kernel(x, edge_index, Wl1, bl1, Wr1, Wl2, bl2, Wr2, Wl3, bl3, Wr3, Wl4, bl4, Wr4, Wc1, bc1, Wc2, bc2):
    raise NotImplementedError("write your pallas kernel here")


# speedup vs baseline: 6.0349x; 6.0349x over previous
import jax
import jax.numpy as jnp
from jax import lax
from jax.experimental import pallas as pl
from jax.experimental.pallas import tpu as pltpu
from jax.experimental.pallas import tpu_sc as plsc

N = 10000
E = 320000
DIN = 128
H = 64
P = 128
CW = 16
NC, NS = 2, 16
NW = NC * NS
EPW = E // NW
K = 80
NITER = EPW // K
RPT = 624
RTAIL = N - NS * RPT
RB = 1000


def _segsum_body(p_hbm, src_hbm, dst_hbm, zeros_hbm, agg_out,
                 src_v, dst_v, rows_v, agg_sh, sem):
    c = lax.axis_index("c")
    s = lax.axis_index("s")
    wid = c * NS + s

    pltpu.sync_copy(src_hbm.at[wid], src_v)
    pltpu.sync_copy(dst_hbm.at[wid], dst_v)
    pltpu.sync_copy(zeros_hbm.at[pl.ds(s * RPT, RPT)],
                    agg_sh.at[pl.ds(s * RPT, RPT)])
    @pl.when(s == NS - 1)
    def _():
        pltpu.sync_copy(zeros_hbm.at[pl.ds(NS * RPT, RTAIL)],
                        agg_sh.at[pl.ds(NS * RPT, RTAIL)])
    plsc.subcore_barrier()

    def body(j, _):
        pltpu.async_copy(p_hbm.at[src_v.at[j]], rows_v, sem).wait()
        pltpu.sync_copy(rows_v, agg_sh.at[dst_v.at[j]], add=True)
        return ()

    lax.fori_loop(0, NITER, body, ())
    plsc.subcore_barrier()

    pltpu.sync_copy(agg_sh.at[pl.ds(s * RPT, RPT)],
                    agg_out.at[pl.ds(c * N + s * RPT, RPT)])
    @pl.when(s == NS - 1)
    def _():
        pltpu.sync_copy(agg_sh.at[pl.ds(NS * RPT, RTAIL)],
                        agg_out.at[pl.ds(c * N + NS * RPT, RTAIL)])


_SC_MESH = plsc.VectorSubcoreMesh(core_axis_name="c", subcore_axis_name="s")

_segsum = pl.kernel(
    _segsum_body,
    out_type=jax.ShapeDtypeStruct((NC * N, P), jnp.float32),
    mesh=_SC_MESH,
    scratch_types=[
        pltpu.VMEM((NITER, K), jnp.int32),
        pltpu.VMEM((NITER, K), jnp.int32),
        pltpu.VMEM((K, P), jnp.float32),
        pltpu.VMEM_SHARED((N, P), jnp.float32),
        pltpu.SemaphoreType.DMA,
    ],
)


def _tc0_body(x_ref, wl_ref, wr_ref, bl_ref, p_ref, r_ref):
    xb = x_ref[...]
    pj = jnp.dot(xb, wl_ref[...], preferred_element_type=jnp.float32)
    p_ref[...] = jnp.concatenate(
        [pj, jnp.ones((RB, P - H), jnp.float32)], axis=1)
    r_ref[...] = (jnp.dot(xb, wr_ref[...], preferred_element_type=jnp.float32)
                  + bl_ref[...])


_tc0 = pl.pallas_call(
    _tc0_body,
    grid=(N // RB,),
    in_specs=[
        pl.BlockSpec((RB, DIN), lambda i: (i, 0)),
        pl.BlockSpec((DIN, H), lambda i: (0, 0)),
        pl.BlockSpec((DIN, H), lambda i: (0, 0)),
        pl.BlockSpec((1, H), lambda i: (0, 0)),
    ],
    out_specs=[
        pl.BlockSpec((RB, P), lambda i: (i, 0)),
        pl.BlockSpec((RB, H), lambda i: (i, 0)),
    ],
    out_shape=[
        jax.ShapeDtypeStruct((N, P), jnp.float32),
        jax.ShapeDtypeStruct((N, H), jnp.float32),
    ],
)


def _tcmid_body(a0_ref, a1_ref, c0_ref, c1_ref, r_ref, wl_ref, wr_ref, bl_ref,
                p_ref, r2_ref):
    cnt = c0_ref[...][:, :1] + c1_ref[...][:, :1]
    inv = 1.0 / jnp.maximum(cnt, 1.0)
    agg = a0_ref[...][:, :H] + a1_ref[...][:, :H]
    h = jnp.maximum(agg * inv + r_ref[...], 0.0)
    pj = jnp.dot(h, wl_ref[...], preferred_element_type=jnp.float32)
    p_ref[...] = jnp.concatenate(
        [pj, jnp.zeros((RB, P - H), jnp.float32)], axis=1)
    r2_ref[...] = (jnp.dot(h, wr_ref[...], preferred_element_type=jnp.float32)
                   + bl_ref[...])


_tcmid = pl.pallas_call(
    _tcmid_body,
    grid=(N // RB,),
    in_specs=[
        pl.BlockSpec((RB, P), lambda i: (i, 0)),
        pl.BlockSpec((RB, P), lambda i: (i, 0)),
        pl.BlockSpec((RB, CW), lambda i: (i, 0)),
        pl.BlockSpec((RB, CW), lambda i: (i, 0)),
        pl.BlockSpec((RB, H), lambda i: (i, 0)),
        pl.BlockSpec((H, H), lambda i: (0, 0)),
        pl.BlockSpec((H, H), lambda i: (0, 0)),
        pl.BlockSpec((1, H), lambda i: (0, 0)),
    ],
    out_specs=[
        pl.BlockSpec((RB, P), lambda i: (i, 0)),
        pl.BlockSpec((RB, H), lambda i: (i, 0)),
    ],
    out_shape=[
        jax.ShapeDtypeStruct((N, P), jnp.float32),
        jax.ShapeDtypeStruct((N, H), jnp.float32),
    ],
)


def _tcfin_body(a0_ref, a1_ref, c0_ref, c1_ref, r_ref, wc1_ref, bc1_ref,
                wc2_ref, bc2_ref, out_ref):
    cnt = c0_ref[...][:, :1] + c1_ref[...][:, :1]
    inv = 1.0 / jnp.maximum(cnt, 1.0)
    agg = a0_ref[...][:, :H] + a1_ref[...][:, :H]
    h = jnp.maximum(agg * inv + r_ref[...], 0.0)
    z = jnp.maximum(jnp.dot(h, wc1_ref[...],
                            preferred_element_type=jnp.float32) + bc1_ref[...],
                    0.0)
    sc = jnp.sum(z * wc2_ref[...], axis=1, keepdims=True) + bc2_ref[...]
    out_ref[...] = 1.0 / (1.0 + jnp.exp(-sc))


_tcfin = pl.pallas_call(
    _tcfin_body,
    grid=(N // RB,),
    in_specs=[
        pl.BlockSpec((RB, P), lambda i: (i, 0)),
        pl.BlockSpec((RB, P), lambda i: (i, 0)),
        pl.BlockSpec((RB, CW), lambda i: (i, 0)),
        pl.BlockSpec((RB, CW), lambda i: (i, 0)),
        pl.BlockSpec((RB, H), lambda i: (i, 0)),
        pl.BlockSpec((H, H // 2), lambda i: (0, 0)),
        pl.BlockSpec((1, H // 2), lambda i: (0, 0)),
        pl.BlockSpec((1, H // 2), lambda i: (0, 0)),
        pl.BlockSpec((1, 1), lambda i: (0, 0)),
    ],
    out_specs=pl.BlockSpec((RB, 1), lambda i: (i, 0)),
    out_shape=jax.ShapeDtypeStruct((N, 1), jnp.float32),
)


def kernel(x, edge_index, Wl1, bl1, Wr1, Wl2, bl2, Wr2, Wl3, bl3, Wr3,
           Wl4, bl4, Wr4, Wc1, bc1, Wc2, bc2):
    src = edge_index[0].reshape(NW, NITER, K)
    dst = edge_index[1].reshape(NW, NITER, K)
    zeros = jnp.zeros((N, P), jnp.float32)

    p1, r1 = _tc0(x, Wl1.T, Wr1.T, bl1.reshape(1, H))
    agg1 = _segsum(p1, src, dst, zeros)
    c0, c1 = agg1[:N, H:H + CW], agg1[N:, H:H + CW]
    p2, r2 = _tcmid(agg1[:N], agg1[N:], c0, c1, r1,
                    Wl2.T, Wr2.T, bl2.reshape(1, H))
    agg2 = _segsum(p2, src, dst, zeros)
    p3, r3 = _tcmid(agg2[:N], agg2[N:], c0, c1, r2,
                    Wl3.T, Wr3.T, bl3.reshape(1, H))
    agg3 = _segsum(p3, src, dst, zeros)
    p4, r4 = _tcmid(agg3[:N], agg3[N:], c0, c1, r3,
                    Wl4.T, Wr4.T, bl4.reshape(1, H))
    agg4 = _segsum(p4, src, dst, zeros)
    score = _tcfin(agg4[:N], agg4[N:], c0, c1, r4,
                   Wc1.T, bc1.reshape(1, H // 2), Wc2, bc2.reshape(1, 1))
    return score.squeeze(-1)


# speedup vs baseline: 7.8013x; 1.2927x over previous
import jax
import jax.numpy as jnp
from jax import lax
from jax.experimental import pallas as pl
from jax.experimental.pallas import tpu as pltpu
from jax.experimental.pallas import tpu_sc as plsc

N = 10000
E = 320000
DIN = 128
H = 64
P = 128
CW = 16
NC, NS = 2, 16
NW = NC * NS
EPW = E // NW
K = 80
NITER = EPW // K
RPT = 624
RTAIL = N - NS * RPT
RB = 1000


def _segsum_body(p_hbm, idx_hbm, zeros_hbm, agg_out,
                 idx0_v, idx1_v, rows0_v, rows1_v, agg_sh,
                 isem0, isem1, gsem0, gsem1):
    c = lax.axis_index("c")
    s = lax.axis_index("s")
    wid = c * NS + s

    pltpu.sync_copy(zeros_hbm.at[pl.ds(s * RPT, RPT)],
                    agg_sh.at[pl.ds(s * RPT, RPT)])
    @pl.when(s == NS - 1)
    def _():
        pltpu.sync_copy(zeros_hbm.at[pl.ds(NS * RPT, RTAIL)],
                        agg_sh.at[pl.ds(NS * RPT, RTAIL)])
    plsc.subcore_barrier()

    def idx(j, buf, sem):
        return pltpu.make_async_copy(idx_hbm.at[wid, j], buf, sem)

    def gather(j, rows, buf, sem):
        return pltpu.make_async_copy(p_hbm.at[buf.at[0]], rows, sem)

    def scatter(j, rows, buf):
        pltpu.sync_copy(rows, agg_sh.at[buf.at[1]], add=True)

    idx(0, idx0_v, isem0).start()
    idx(1, idx1_v, isem1).start()
    idx(0, idx0_v, isem0).wait()
    gather(0, rows0_v, idx0_v, gsem0).start()
    idx(1, idx1_v, isem1).wait()
    gather(1, rows1_v, idx1_v, gsem1).start()

    def step(j, rows, buf, isem, gsem):
        gather(j, rows, buf, gsem).wait()
        scatter(j, rows, buf)
        @pl.when(j + 2 < NITER)
        def _():
            idx(j + 2, buf, isem).start()
            idx(j + 2, buf, isem).wait()
            gather(j + 2, rows, buf, gsem).start()

    def body(j, _):
        @pl.when(j % 2 == 0)
        def _():
            step(j, rows0_v, idx0_v, isem0, gsem0)
        @pl.when(j % 2 == 1)
        def _():
            step(j, rows1_v, idx1_v, isem1, gsem1)
        return ()

    lax.fori_loop(0, NITER, body, ())
    plsc.subcore_barrier()

    pltpu.sync_copy(agg_sh.at[pl.ds(s * RPT, RPT)],
                    agg_out.at[pl.ds(c * N + s * RPT, RPT)])
    @pl.when(s == NS - 1)
    def _():
        pltpu.sync_copy(agg_sh.at[pl.ds(NS * RPT, RTAIL)],
                        agg_out.at[pl.ds(c * N + NS * RPT, RTAIL)])


_SC_MESH = plsc.VectorSubcoreMesh(core_axis_name="c", subcore_axis_name="s")

_segsum = pl.kernel(
    _segsum_body,
    out_type=jax.ShapeDtypeStruct((NC * N, P), jnp.float32),
    mesh=_SC_MESH,
    scratch_types=[
        pltpu.VMEM((2, K), jnp.int32),
        pltpu.VMEM((2, K), jnp.int32),
        pltpu.VMEM((K, P), jnp.float32),
        pltpu.VMEM((K, P), jnp.float32),
        pltpu.VMEM_SHARED((N, P), jnp.float32),
        pltpu.SemaphoreType.DMA,
        pltpu.SemaphoreType.DMA,
        pltpu.SemaphoreType.DMA,
        pltpu.SemaphoreType.DMA,
    ],
)


def _tc0_body(x_ref, wl_ref, wr_ref, bl_ref, p_ref, r_ref):
    xb = x_ref[...]
    pj = jnp.dot(xb, wl_ref[...], preferred_element_type=jnp.float32)
    p_ref[...] = jnp.concatenate(
        [pj, jnp.ones((RB, P - H), jnp.float32)], axis=1)
    r_ref[...] = (jnp.dot(xb, wr_ref[...], preferred_element_type=jnp.float32)
                  + bl_ref[...])


_tc0 = pl.pallas_call(
    _tc0_body,
    grid=(N // RB,),
    in_specs=[
        pl.BlockSpec((RB, DIN), lambda i: (i, 0)),
        pl.BlockSpec((DIN, H), lambda i: (0, 0)),
        pl.BlockSpec((DIN, H), lambda i: (0, 0)),
        pl.BlockSpec((1, H), lambda i: (0, 0)),
    ],
    out_specs=[
        pl.BlockSpec((RB, P), lambda i: (i, 0)),
        pl.BlockSpec((RB, H), lambda i: (i, 0)),
    ],
    out_shape=[
        jax.ShapeDtypeStruct((N, P), jnp.float32),
        jax.ShapeDtypeStruct((N, H), jnp.float32),
    ],
)


def _tcmid_body(a0_ref, a1_ref, c0_ref, c1_ref, r_ref, wl_ref, wr_ref, bl_ref,
                p_ref, r2_ref):
    cnt = c0_ref[...][:, :1] + c1_ref[...][:, :1]
    inv = 1.0 / jnp.maximum(cnt, 1.0)
    agg = a0_ref[...][:, :H] + a1_ref[...][:, :H]
    h = jnp.maximum(agg * inv + r_ref[...], 0.0)
    pj = jnp.dot(h, wl_ref[...], preferred_element_type=jnp.float32)
    p_ref[...] = jnp.concatenate(
        [pj, jnp.zeros((RB, P - H), jnp.float32)], axis=1)
    r2_ref[...] = (jnp.dot(h, wr_ref[...], preferred_element_type=jnp.float32)
                   + bl_ref[...])


_tcmid = pl.pallas_call(
    _tcmid_body,
    grid=(N // RB,),
    in_specs=[
        pl.BlockSpec((RB, P), lambda i: (i, 0)),
        pl.BlockSpec((RB, P), lambda i: (i, 0)),
        pl.BlockSpec((RB, CW), lambda i: (i, 0)),
        pl.BlockSpec((RB, CW), lambda i: (i, 0)),
        pl.BlockSpec((RB, H), lambda i: (i, 0)),
        pl.BlockSpec((H, H), lambda i: (0, 0)),
        pl.BlockSpec((H, H), lambda i: (0, 0)),
        pl.BlockSpec((1, H), lambda i: (0, 0)),
    ],
    out_specs=[
        pl.BlockSpec((RB, P), lambda i: (i, 0)),
        pl.BlockSpec((RB, H), lambda i: (i, 0)),
    ],
    out_shape=[
        jax.ShapeDtypeStruct((N, P), jnp.float32),
        jax.ShapeDtypeStruct((N, H), jnp.float32),
    ],
)


def _tcfin_body(a0_ref, a1_ref, c0_ref, c1_ref, r_ref, wc1_ref, bc1_ref,
                wc2_ref, bc2_ref, out_ref):
    cnt = c0_ref[...][:, :1] + c1_ref[...][:, :1]
    inv = 1.0 / jnp.maximum(cnt, 1.0)
    agg = a0_ref[...][:, :H] + a1_ref[...][:, :H]
    h = jnp.maximum(agg * inv + r_ref[...], 0.0)
    z = jnp.maximum(jnp.dot(h, wc1_ref[...],
                            preferred_element_type=jnp.float32) + bc1_ref[...],
                    0.0)
    sc = jnp.sum(z * wc2_ref[...], axis=1, keepdims=True) + bc2_ref[...]
    out_ref[...] = 1.0 / (1.0 + jnp.exp(-sc))


_tcfin = pl.pallas_call(
    _tcfin_body,
    grid=(N // RB,),
    in_specs=[
        pl.BlockSpec((RB, P), lambda i: (i, 0)),
        pl.BlockSpec((RB, P), lambda i: (i, 0)),
        pl.BlockSpec((RB, CW), lambda i: (i, 0)),
        pl.BlockSpec((RB, CW), lambda i: (i, 0)),
        pl.BlockSpec((RB, H), lambda i: (i, 0)),
        pl.BlockSpec((H, H // 2), lambda i: (0, 0)),
        pl.BlockSpec((1, H // 2), lambda i: (0, 0)),
        pl.BlockSpec((1, H // 2), lambda i: (0, 0)),
        pl.BlockSpec((1, 1), lambda i: (0, 0)),
    ],
    out_specs=pl.BlockSpec((RB, 1), lambda i: (i, 0)),
    out_shape=jax.ShapeDtypeStruct((N, 1), jnp.float32),
)


def kernel(x, edge_index, Wl1, bl1, Wr1, Wl2, bl2, Wr2, Wl3, bl3, Wr3,
           Wl4, bl4, Wr4, Wc1, bc1, Wc2, bc2):
    idx = jnp.stack([edge_index[0].reshape(NW, NITER, K),
                     edge_index[1].reshape(NW, NITER, K)], axis=2)
    zeros = jnp.zeros((N, P), jnp.float32)

    p1, r1 = _tc0(x, Wl1.T, Wr1.T, bl1.reshape(1, H))
    agg1 = _segsum(p1, idx, zeros)
    c0, c1 = agg1[:N, H:H + CW], agg1[N:, H:H + CW]
    p2, r2 = _tcmid(agg1[:N], agg1[N:], c0, c1, r1,
                    Wl2.T, Wr2.T, bl2.reshape(1, H))
    agg2 = _segsum(p2, idx, zeros)
    p3, r3 = _tcmid(agg2[:N], agg2[N:], c0, c1, r2,
                    Wl3.T, Wr3.T, bl3.reshape(1, H))
    agg3 = _segsum(p3, idx, zeros)
    p4, r4 = _tcmid(agg3[:N], agg3[N:], c0, c1, r3,
                    Wl4.T, Wr4.T, bl4.reshape(1, H))
    agg4 = _segsum(p4, idx, zeros)
    score = _tcfin(agg4[:N], agg4[N:], c0, c1, r4,
                   Wc1.T, bc1.reshape(1, H // 2), Wc2, bc2.reshape(1, 1))
    return score.squeeze(-1)


# speedup vs baseline: 9.1172x; 1.1687x over previous
import jax
import jax.numpy as jnp
from jax import lax
from jax.experimental import pallas as pl
from jax.experimental.pallas import tpu as pltpu
from jax.experimental.pallas import tpu_sc as plsc

N = 10000
E = 320000
DIN = 128
H = 64
P = 128
CW = 16
NC, NS = 2, 16
NW = NC * NS
EPW = E // NW
K = 125
NITER = EPW // K
RPT = 624
RTAIL = N - NS * RPT
RB = 1000


def _segsum_body(p_hbm, idx_hbm, zeros_hbm, agg_out,
                 idx0_v, idx1_v, rows0_v, rows1_v, agg_sh,
                 isem0, isem1, gsem0, gsem1):
    c = lax.axis_index("c")
    s = lax.axis_index("s")
    wid = c * NS + s

    pltpu.sync_copy(zeros_hbm.at[pl.ds(s * RPT, RPT)],
                    agg_sh.at[pl.ds(s * RPT, RPT)])
    @pl.when(s == NS - 1)
    def _():
        pltpu.sync_copy(zeros_hbm.at[pl.ds(NS * RPT, RTAIL)],
                        agg_sh.at[pl.ds(NS * RPT, RTAIL)])
    plsc.subcore_barrier()

    def idx(j, buf, sem):
        return pltpu.make_async_copy(idx_hbm.at[wid, j], buf, sem)

    def gather(j, rows, buf, sem):
        return pltpu.make_async_copy(p_hbm.at[buf.at[0]], rows, sem)

    def scatter(j, rows, buf):
        pltpu.sync_copy(rows, agg_sh.at[buf.at[1]], add=True)

    idx(0, idx0_v, isem0).start()
    idx(1, idx1_v, isem1).start()
    idx(0, idx0_v, isem0).wait()
    gather(0, rows0_v, idx0_v, gsem0).start()
    idx(1, idx1_v, isem1).wait()
    gather(1, rows1_v, idx1_v, gsem1).start()

    def step(j, rows, buf, isem, gsem):
        gather(j, rows, buf, gsem).wait()
        scatter(j, rows, buf)
        @pl.when(j + 2 < NITER)
        def _():
            idx(j + 2, buf, isem).start()
            idx(j + 2, buf, isem).wait()
            gather(j + 2, rows, buf, gsem).start()

    def body(j, _):
        @pl.when(j % 2 == 0)
        def _():
            step(j, rows0_v, idx0_v, isem0, gsem0)
        @pl.when(j % 2 == 1)
        def _():
            step(j, rows1_v, idx1_v, isem1, gsem1)
        return ()

    lax.fori_loop(0, NITER, body, ())
    plsc.subcore_barrier()

    pltpu.sync_copy(agg_sh.at[pl.ds(s * RPT, RPT)],
                    agg_out.at[pl.ds(c * N + s * RPT, RPT)])
    @pl.when(s == NS - 1)
    def _():
        pltpu.sync_copy(agg_sh.at[pl.ds(NS * RPT, RTAIL)],
                        agg_out.at[pl.ds(c * N + NS * RPT, RTAIL)])


_SC_MESH = plsc.VectorSubcoreMesh(core_axis_name="c", subcore_axis_name="s")

_segsum = pl.kernel(
    _segsum_body,
    out_type=jax.ShapeDtypeStruct((NC * N, P), jnp.float32),
    mesh=_SC_MESH,
    scratch_types=[
        pltpu.VMEM((2, K), jnp.int32),
        pltpu.VMEM((2, K), jnp.int32),
        pltpu.VMEM((K, P), jnp.float32),
        pltpu.VMEM((K, P), jnp.float32),
        pltpu.VMEM_SHARED((N, P), jnp.float32),
        pltpu.SemaphoreType.DMA,
        pltpu.SemaphoreType.DMA,
        pltpu.SemaphoreType.DMA,
        pltpu.SemaphoreType.DMA,
    ],
)


def _tc0_body(x_ref, wl_ref, wr_ref, bl_ref, p_ref, r_ref):
    xb = x_ref[...]
    pj = jnp.dot(xb, wl_ref[...], preferred_element_type=jnp.float32)
    p_ref[...] = jnp.concatenate(
        [pj, jnp.ones((RB, P - H), jnp.float32)], axis=1)
    r_ref[...] = (jnp.dot(xb, wr_ref[...], preferred_element_type=jnp.float32)
                  + bl_ref[...])


_tc0 = pl.pallas_call(
    _tc0_body,
    grid=(N // RB,),
    in_specs=[
        pl.BlockSpec((RB, DIN), lambda i: (i, 0)),
        pl.BlockSpec((DIN, H), lambda i: (0, 0)),
        pl.BlockSpec((DIN, H), lambda i: (0, 0)),
        pl.BlockSpec((1, H), lambda i: (0, 0)),
    ],
    out_specs=[
        pl.BlockSpec((RB, P), lambda i: (i, 0)),
        pl.BlockSpec((RB, H), lambda i: (i, 0)),
    ],
    out_shape=[
        jax.ShapeDtypeStruct((N, P), jnp.float32),
        jax.ShapeDtypeStruct((N, H), jnp.float32),
    ],
)


def _tcmid_body(a0_ref, a1_ref, c0_ref, c1_ref, r_ref, wl_ref, wr_ref, bl_ref,
                p_ref, r2_ref):
    cnt = c0_ref[...][:, :1] + c1_ref[...][:, :1]
    inv = 1.0 / jnp.maximum(cnt, 1.0)
    agg = a0_ref[...][:, :H] + a1_ref[...][:, :H]
    h = jnp.maximum(agg * inv + r_ref[...], 0.0)
    pj = jnp.dot(h, wl_ref[...], preferred_element_type=jnp.float32)
    p_ref[...] = jnp.concatenate(
        [pj, jnp.zeros((RB, P - H), jnp.float32)], axis=1)
    r2_ref[...] = (jnp.dot(h, wr_ref[...], preferred_element_type=jnp.float32)
                   + bl_ref[...])


_tcmid = pl.pallas_call(
    _tcmid_body,
    grid=(N // RB,),
    in_specs=[
        pl.BlockSpec((RB, P), lambda i: (i, 0)),
        pl.BlockSpec((RB, P), lambda i: (i, 0)),
        pl.BlockSpec((RB, CW), lambda i: (i, 0)),
        pl.BlockSpec((RB, CW), lambda i: (i, 0)),
        pl.BlockSpec((RB, H), lambda i: (i, 0)),
        pl.BlockSpec((H, H), lambda i: (0, 0)),
        pl.BlockSpec((H, H), lambda i: (0, 0)),
        pl.BlockSpec((1, H), lambda i: (0, 0)),
    ],
    out_specs=[
        pl.BlockSpec((RB, P), lambda i: (i, 0)),
        pl.BlockSpec((RB, H), lambda i: (i, 0)),
    ],
    out_shape=[
        jax.ShapeDtypeStruct((N, P), jnp.float32),
        jax.ShapeDtypeStruct((N, H), jnp.float32),
    ],
)


def _tcfin_body(a0_ref, a1_ref, c0_ref, c1_ref, r_ref, wc1_ref, bc1_ref,
                wc2_ref, bc2_ref, out_ref):
    cnt = c0_ref[...][:, :1] + c1_ref[...][:, :1]
    inv = 1.0 / jnp.maximum(cnt, 1.0)
    agg = a0_ref[...][:, :H] + a1_ref[...][:, :H]
    h = jnp.maximum(agg * inv + r_ref[...], 0.0)
    z = jnp.maximum(jnp.dot(h, wc1_ref[...],
                            preferred_element_type=jnp.float32) + bc1_ref[...],
                    0.0)
    sc = jnp.sum(z * wc2_ref[...], axis=1, keepdims=True) + bc2_ref[...]
    out_ref[...] = 1.0 / (1.0 + jnp.exp(-sc))


_tcfin = pl.pallas_call(
    _tcfin_body,
    grid=(N // RB,),
    in_specs=[
        pl.BlockSpec((RB, P), lambda i: (i, 0)),
        pl.BlockSpec((RB, P), lambda i: (i, 0)),
        pl.BlockSpec((RB, CW), lambda i: (i, 0)),
        pl.BlockSpec((RB, CW), lambda i: (i, 0)),
        pl.BlockSpec((RB, H), lambda i: (i, 0)),
        pl.BlockSpec((H, H // 2), lambda i: (0, 0)),
        pl.BlockSpec((1, H // 2), lambda i: (0, 0)),
        pl.BlockSpec((1, H // 2), lambda i: (0, 0)),
        pl.BlockSpec((1, 1), lambda i: (0, 0)),
    ],
    out_specs=pl.BlockSpec((RB, 1), lambda i: (i, 0)),
    out_shape=jax.ShapeDtypeStruct((N, 1), jnp.float32),
)


def kernel(x, edge_index, Wl1, bl1, Wr1, Wl2, bl2, Wr2, Wl3, bl3, Wr3,
           Wl4, bl4, Wr4, Wc1, bc1, Wc2, bc2):
    idx = jnp.stack([edge_index[0].reshape(NW, NITER, K),
                     edge_index[1].reshape(NW, NITER, K)], axis=2)
    zeros = jnp.zeros((N, P), jnp.float32)

    p1, r1 = _tc0(x, Wl1.T, Wr1.T, bl1.reshape(1, H))
    agg1 = _segsum(p1, idx, zeros)
    c0, c1 = agg1[:N, H:H + CW], agg1[N:, H:H + CW]
    p2, r2 = _tcmid(agg1[:N], agg1[N:], c0, c1, r1,
                    Wl2.T, Wr2.T, bl2.reshape(1, H))
    agg2 = _segsum(p2, idx, zeros)
    p3, r3 = _tcmid(agg2[:N], agg2[N:], c0, c1, r2,
                    Wl3.T, Wr3.T, bl3.reshape(1, H))
    agg3 = _segsum(p3, idx, zeros)
    p4, r4 = _tcmid(agg3[:N], agg3[N:], c0, c1, r3,
                    Wl4.T, Wr4.T, bl4.reshape(1, H))
    agg4 = _segsum(p4, idx, zeros)
    score = _tcfin(agg4[:N], agg4[N:], c0, c1, r4,
                   Wc1.T, bc1.reshape(1, H // 2), Wc2, bc2.reshape(1, 1))
    return score.squeeze(-1)
